# baseline (speedup 1.0000x reference)
import functools

import jax
import jax.numpy as jnp
from jax import lax
from jax.experimental import pallas as pl
from jax.experimental.pallas import tpu as pltpu
from jax.experimental.pallas import tpu_sc as plsc

VOC = 1000000
EMB = 64
B = 4096
T = 20

NC = 2
NS = 16
NW = NC * NS
BPW = B // NW
CB = 4
ROWS = CB * T
NCHUNK = BPW // CB
LANES = 16
EC = EMB // LANES

RBLK = 512
NRBLK = (VOC + RBLK - 1) // RBLK
LINR = NRBLK * RBLK // 2
EMBR = VOC // 2


def _tc_relayout(lin_t):
  def body(a_ref, o_ref):
    ident = jnp.eye(EMB, dtype=jnp.float32)
    t = lax.dot_general(a_ref[...], ident, (((0,), (0,)), ((), ())),
                        preferred_element_type=jnp.float32)
    o_ref[:, 0:EMB] = t[0:RBLK // 2, :]
    o_ref[:, EMB:2 * EMB] = t[RBLK // 2:RBLK, :]

  return pl.pallas_call(
      body,
      grid=(NRBLK,),
      in_specs=[pl.BlockSpec((EMB, RBLK), lambda i: (0, i))],
      out_specs=pl.BlockSpec((RBLK // 2, 128), lambda i: (i, 0)),
      out_shape=jax.ShapeDtypeStruct((LINR, 128), jnp.float32),
  )(lin_t)


def _sc_logits(iphys, epar_bt, tphys, tpar, rphys, rpar, emb_r, lin_r):
  mesh = plsc.VectorSubcoreMesh(core_axis_name="c", subcore_axis_name="s")

  @functools.partial(
      pl.kernel,
      out_type=[
          jax.ShapeDtypeStruct((B * T,), jnp.float32),
          jax.ShapeDtypeStruct((B * T,), jnp.float32),
      ],
      mesh=mesh,
      scratch_types=[
          pltpu.VMEM((BPW,), jnp.int32),
          pltpu.VMEM((BPW * T,), jnp.int32),
          pltpu.VMEM((BPW, 128), jnp.float32),
          pltpu.VMEM((BPW * T,), jnp.int32),
          pltpu.VMEM((BPW * T,), jnp.int32),
          pltpu.VMEM((BPW * T,), jnp.int32),
          pltpu.VMEM((BPW * T,), jnp.int32),
          pltpu.VMEM((2, ROWS, 128), jnp.float32),
          pltpu.VMEM((BPW * T,), jnp.float32),
          pltpu.VMEM((BPW * T,), jnp.float32),
          pltpu.SemaphoreType.DMA,
          pltpu.SemaphoreType.DMA,
          pltpu.SemaphoreType.DMA,
      ],
  )
  def k(iphys_h, epar_h, tphys_h, tpar_h, rphys_h, rpar_h, emb_h, lin_h,
        pos_h, neg_h,
        iidx, eparv, embv, tidx, tparv, ridx, rparv, rows2, posv, negv,
        sem0, sem1, sem_e):
    wid = lax.axis_index("s") * NC + lax.axis_index("c")
    base = wid * BPW

    pltpu.sync_copy(iphys_h.at[pl.ds(base, BPW)], iidx)
    emb_cp = pltpu.async_copy(emb_h.at[iidx], embv, sem_e)
    pltpu.sync_copy(epar_h.at[pl.ds(base * T, BPW * T)], eparv)
    pltpu.sync_copy(tphys_h.at[pl.ds(base * T, BPW * T)], tidx)
    pltpu.sync_copy(tpar_h.at[pl.ds(base * T, BPW * T)], tparv)
    pltpu.sync_copy(rphys_h.at[pl.ds(base * T, BPW * T)], ridx)
    pltpu.sync_copy(rpar_h.at[pl.ds(base * T, BPW * T)], rparv)
    emb_cp.wait()

    sems = (sem0, sem1)

    def start(idxref, g, slot):
      pltpu.async_copy(
          lin_h.at[idxref.at[pl.ds(g * ROWS, ROWS)]],
          rows2.at[slot], sems[slot])

    def wait(idxref, g, slot):
      pltpu.make_async_copy(
          lin_h.at[idxref.at[pl.ds(g * ROWS, ROWS)]],
          rows2.at[slot], sems[slot]).wait()

    lane_masks = [lax.iota(jnp.int32, LANES) == j for j in range(LANES)]
    perms = [lax.iota(jnp.int32, LANES) ^ sh for sh in (8, 4, 2, 1)]

    def compute(parref, outflat, g, slot):
      res_lo = jnp.zeros((LANES,), jnp.float32)
      res_hi = jnp.zeros((LANES,), jnp.float32)
      e = None
      pv = None
      for d in range(ROWS):
        cb, t = divmod(d, T)
        if t == 0:
          b_local = g * CB + cb
          e = [embv[b_local, pl.ds(c * LANES, LANES)] for c in range(2 * EC)]
        if d % LANES == 0:
          pv = parref[pl.ds(g * ROWS + d, LANES)]
        cpar = pv[d % LANES]
        coff = cpar * EMB
        acc_lo = None
        acc_hi = None
        for c in range(EC):
          ctx = rows2[slot, d, pl.ds(coff + c * LANES, LANES)]
          pl_ = ctx * e[c]
          ph_ = ctx * e[EC + c]
          acc_lo = pl_ if acc_lo is None else acc_lo + pl_
          acc_hi = ph_ if acc_hi is None else acc_hi + ph_
        for p in perms:
          acc_lo = acc_lo + jnp.take(acc_lo, p)
          acc_hi = acc_hi + jnp.take(acc_hi, p)
        res_lo = jnp.where(lane_masks[d % LANES], acc_lo, res_lo)
        res_hi = jnp.where(lane_masks[d % LANES], acc_hi, res_hi)
        if d % LANES == LANES - 1:
          epv = eparv[pl.ds(g * ROWS + (d - LANES + 1), LANES)]
          res = jnp.where(epv == 0, res_lo, res_hi)
          outflat[pl.ds(g * ROWS + (d - LANES + 1), LANES)] = res

    def run_table(idxref, parref, outref):
      start(idxref, 0, 0)

      def body(i, carry):
        g0 = 2 * i

        @pl.when(g0 + 1 < NCHUNK)
        def _():
          start(idxref, g0 + 1, 1)

        wait(idxref, g0, 0)
        compute(parref, outref, g0, 0)

        @pl.when(g0 + 2 < NCHUNK)
        def _():
          start(idxref, g0 + 2, 0)

        @pl.when(g0 + 1 < NCHUNK)
        def _():
          wait(idxref, g0 + 1, 1)
          compute(parref, outref, g0 + 1, 1)

        return carry

      lax.fori_loop(0, NCHUNK // 2, body, 0)

    run_table(tidx, tparv, posv)
    run_table(ridx, rparv, negv)

    pltpu.sync_copy(posv, pos_h.at[pl.ds(base * T, BPW * T)])
    pltpu.sync_copy(negv, neg_h.at[pl.ds(base * T, BPW * T)])

  return k(iphys, epar_bt, tphys, tpar, rphys, rpar, emb_r, lin_r)


def _tc_loss(pos, neg):
  def body(pos_ref, neg_ref, o_ref):
    p = jax.nn.sigmoid(pos_ref[...])
    n = jax.nn.sigmoid(neg_ref[...])
    pst = -jnp.mean(jnp.log(p))
    ngt = -jnp.mean(jnp.log(1.0 - n + 1e-3))
    o_ref[0, 0] = pst + ngt

  out = pl.pallas_call(
      body,
      out_shape=jax.ShapeDtypeStruct((1, 1), jnp.float32),
      in_specs=[
          pl.BlockSpec(memory_space=pltpu.VMEM),
          pl.BlockSpec(memory_space=pltpu.VMEM),
      ],
      out_specs=pl.BlockSpec(memory_space=pltpu.SMEM),
  )(pos, neg)
  return out[0, 0]


def kernel(inpt, trgs, rand, emb_table, lin_w):
  inpt = inpt.astype(jnp.int32)
  trgs = trgs.astype(jnp.int32)
  rand = rand.astype(jnp.int32)

  emb_r = emb_table.reshape(EMBR, 128)
  iphys = inpt >> 1
  epar_bt = jnp.broadcast_to((inpt & 1)[:, None], (B, T)).reshape(-1)

  lin_r = _tc_relayout(lin_w.T)
  tphys = (((trgs >> 9) << 8) | (trgs & 255)).reshape(-1)
  tpar = ((trgs >> 8) & 1).reshape(-1)
  rphys = (((rand >> 9) << 8) | (rand & 255)).reshape(-1)
  rpar = ((rand >> 8) & 1).reshape(-1)

  pos, neg = _sc_logits(iphys, epar_bt, tphys, tpar, rphys, rpar,
                        emb_r, lin_r)
  return _tc_loss(pos.reshape(B * T // 128, 128),
                  neg.reshape(B * T // 128, 128))


# speedup vs baseline: 1.9956x; 1.9956x over previous
import functools

import jax
import jax.numpy as jnp
from jax import lax
from jax.experimental import pallas as pl
from jax.experimental.pallas import tpu as pltpu
from jax.experimental.pallas import tpu_sc as plsc

VOC = 1000000
EMB = 64
B = 4096
T = 20

NC = 2
NS = 16
NW = NC * NS
BPW = B // NW
CB = 4
ROWS = CB * T
NCHUNK = BPW // CB
LANES = 16
EC = EMB // LANES

RBLK = 4096
NRBLK = (VOC + RBLK - 1) // RBLK
LINR = NRBLK * RBLK // 2
EMBR = VOC // 2


def _tc_relayout(lin_t):
  def body(a_ref, o_ref):
    ident = jnp.eye(EMB, dtype=jnp.float32)
    t = lax.dot_general(a_ref[...], ident, (((0,), (0,)), ((), ())),
                        preferred_element_type=jnp.float32)
    o_ref[:, 0:EMB] = t[0:RBLK // 2, :]
    o_ref[:, EMB:2 * EMB] = t[RBLK // 2:RBLK, :]

  return pl.pallas_call(
      body,
      grid=(NRBLK,),
      in_specs=[pl.BlockSpec((EMB, RBLK), lambda i: (0, i))],
      out_specs=pl.BlockSpec((RBLK // 2, 128), lambda i: (i, 0)),
      out_shape=jax.ShapeDtypeStruct((LINR, 128), jnp.float32),
  )(lin_t)


def _sc_logits(iphys, epar_bt, tphys, tpar, rphys, rpar, emb_r, lin_r):
  mesh = plsc.VectorSubcoreMesh(core_axis_name="c", subcore_axis_name="s")

  @functools.partial(
      pl.kernel,
      out_type=[
          jax.ShapeDtypeStruct((B * T,), jnp.float32),
          jax.ShapeDtypeStruct((B * T,), jnp.float32),
      ],
      mesh=mesh,
      scratch_types=[
          pltpu.VMEM((BPW,), jnp.int32),
          pltpu.VMEM((BPW * T,), jnp.int32),
          pltpu.VMEM((BPW, 128), jnp.float32),
          pltpu.VMEM((BPW * T,), jnp.int32),
          pltpu.VMEM((BPW * T,), jnp.int32),
          pltpu.VMEM((BPW * T,), jnp.int32),
          pltpu.VMEM((BPW * T,), jnp.int32),
          pltpu.VMEM((2, ROWS, 128), jnp.float32),
          pltpu.VMEM((BPW * T,), jnp.float32),
          pltpu.VMEM((BPW * T,), jnp.float32),
          pltpu.SemaphoreType.DMA,
          pltpu.SemaphoreType.DMA,
          pltpu.SemaphoreType.DMA,
      ],
  )
  def k(iphys_h, epar_h, tphys_h, tpar_h, rphys_h, rpar_h, emb_h, lin_h,
        pos_h, neg_h,
        iidx, eparv, embv, tidx, tparv, ridx, rparv, rows2, posv, negv,
        sem0, sem1, sem_e):
    wid = lax.axis_index("s") * NC + lax.axis_index("c")
    base = wid * BPW

    pltpu.sync_copy(iphys_h.at[pl.ds(base, BPW)], iidx)
    emb_cp = pltpu.async_copy(emb_h.at[iidx], embv, sem_e)
    pltpu.sync_copy(epar_h.at[pl.ds(base * T, BPW * T)], eparv)
    pltpu.sync_copy(tphys_h.at[pl.ds(base * T, BPW * T)], tidx)
    pltpu.sync_copy(tpar_h.at[pl.ds(base * T, BPW * T)], tparv)
    pltpu.sync_copy(rphys_h.at[pl.ds(base * T, BPW * T)], ridx)
    pltpu.sync_copy(rpar_h.at[pl.ds(base * T, BPW * T)], rparv)
    emb_cp.wait()

    sems = (sem0, sem1)

    def start(idxref, g, slot):
      pltpu.async_copy(
          lin_h.at[idxref.at[pl.ds(g * ROWS, ROWS)]],
          rows2.at[slot], sems[slot])

    def wait(idxref, g, slot):
      pltpu.make_async_copy(
          lin_h.at[idxref.at[pl.ds(g * ROWS, ROWS)]],
          rows2.at[slot], sems[slot]).wait()

    lane_masks = [lax.iota(jnp.int32, LANES) == j for j in range(LANES)]
    perms = [lax.iota(jnp.int32, LANES) ^ sh for sh in (8, 4, 2, 1)]

    def compute(parref, outflat, g, slot):
      res_lo = jnp.zeros((LANES,), jnp.float32)
      res_hi = jnp.zeros((LANES,), jnp.float32)
      e = None
      pv = None
      for d in range(ROWS):
        cb, t = divmod(d, T)
        if t == 0:
          b_local = g * CB + cb
          e = [embv[b_local, pl.ds(c * LANES, LANES)] for c in range(2 * EC)]
        if d % LANES == 0:
          pv = parref[pl.ds(g * ROWS + d, LANES)]
        cpar = pv[d % LANES]
        coff = cpar * EMB
        acc_lo = None
        acc_hi = None
        for c in range(EC):
          ctx = rows2[slot, d, pl.ds(coff + c * LANES, LANES)]
          pl_ = ctx * e[c]
          ph_ = ctx * e[EC + c]
          acc_lo = pl_ if acc_lo is None else acc_lo + pl_
          acc_hi = ph_ if acc_hi is None else acc_hi + ph_
        for p in perms:
          acc_lo = acc_lo + jnp.take(acc_lo, p)
          acc_hi = acc_hi + jnp.take(acc_hi, p)
        res_lo = jnp.where(lane_masks[d % LANES], acc_lo, res_lo)
        res_hi = jnp.where(lane_masks[d % LANES], acc_hi, res_hi)
        if d % LANES == LANES - 1:
          epv = eparv[pl.ds(g * ROWS + (d - LANES + 1), LANES)]
          res = jnp.where(epv == 0, res_lo, res_hi)
          outflat[pl.ds(g * ROWS + (d - LANES + 1), LANES)] = res

    def run_table(idxref, parref, outref):
      start(idxref, 0, 0)

      def body(i, carry):
        g0 = 2 * i

        @pl.when(g0 + 1 < NCHUNK)
        def _():
          start(idxref, g0 + 1, 1)

        wait(idxref, g0, 0)
        compute(parref, outref, g0, 0)

        @pl.when(g0 + 2 < NCHUNK)
        def _():
          start(idxref, g0 + 2, 0)

        @pl.when(g0 + 1 < NCHUNK)
        def _():
          wait(idxref, g0 + 1, 1)
          compute(parref, outref, g0 + 1, 1)

        return carry

      lax.fori_loop(0, NCHUNK // 2, body, 0)

    run_table(tidx, tparv, posv)
    run_table(ridx, rparv, negv)

    pltpu.sync_copy(posv, pos_h.at[pl.ds(base * T, BPW * T)])
    pltpu.sync_copy(negv, neg_h.at[pl.ds(base * T, BPW * T)])

  return k(iphys, epar_bt, tphys, tpar, rphys, rpar, emb_r, lin_r)


def _tc_loss(pos, neg):
  def body(pos_ref, neg_ref, o_ref):
    p = jax.nn.sigmoid(pos_ref[...])
    n = jax.nn.sigmoid(neg_ref[...])
    pst = -jnp.mean(jnp.log(p))
    ngt = -jnp.mean(jnp.log(1.0 - n + 1e-3))
    o_ref[0, 0] = pst + ngt

  out = pl.pallas_call(
      body,
      out_shape=jax.ShapeDtypeStruct((1, 1), jnp.float32),
      in_specs=[
          pl.BlockSpec(memory_space=pltpu.VMEM),
          pl.BlockSpec(memory_space=pltpu.VMEM),
      ],
      out_specs=pl.BlockSpec(memory_space=pltpu.SMEM),
  )(pos, neg)
  return out[0, 0]


def kernel(inpt, trgs, rand, emb_table, lin_w):
  inpt = inpt.astype(jnp.int32)
  trgs = trgs.astype(jnp.int32)
  rand = rand.astype(jnp.int32)

  emb_r = emb_table.reshape(EMBR, 128)
  iphys = inpt >> 1
  epar_bt = jnp.broadcast_to((inpt & 1)[:, None], (B, T)).reshape(-1)

  lin_r = _tc_relayout(lin_w.T)
  tphys = (((trgs >> 12) << 11) | (trgs & 2047)).reshape(-1)
  tpar = ((trgs >> 11) & 1).reshape(-1)
  rphys = (((rand >> 12) << 11) | (rand & 2047)).reshape(-1)
  rpar = ((rand >> 11) & 1).reshape(-1)

  pos, neg = _sc_logits(iphys, epar_bt, tphys, tpar, rphys, rpar,
                        emb_r, lin_r)
  return _tc_loss(pos.reshape(B * T // 128, 128),
                  neg.reshape(B * T // 128, 128))


# speedup vs baseline: 2.2766x; 1.1408x over previous
import functools

import jax
import jax.numpy as jnp
from jax import lax
from jax.experimental import pallas as pl
from jax.experimental.pallas import tpu as pltpu
from jax.experimental.pallas import tpu_sc as plsc

VOC = 1000000
EMB = 64
B = 4096
T = 20

NC = 2
NS = 16
NW = NC * NS
BPW = B // NW
CB = 4
ROWS = CB * T
NCHUNK = BPW // CB
LANES = 16
EC = EMB // LANES

RBLK = 8192
NRBLK = (VOC + RBLK - 1) // RBLK
LINR = NRBLK * RBLK // 2
EMBR = VOC // 2


def _tc_relayout(lin_t):
  def body(a_ref, o_ref):
    ident = jnp.eye(EMB, dtype=jnp.float32)
    t = lax.dot_general(a_ref[...], ident, (((0,), (0,)), ((), ())),
                        preferred_element_type=jnp.float32)
    o_ref[:, 0:EMB] = t[0:RBLK // 2, :]
    o_ref[:, EMB:2 * EMB] = t[RBLK // 2:RBLK, :]

  return pl.pallas_call(
      body,
      grid=(NRBLK,),
      in_specs=[pl.BlockSpec((EMB, RBLK), lambda i: (0, i))],
      out_specs=pl.BlockSpec((RBLK // 2, 128), lambda i: (i, 0)),
      out_shape=jax.ShapeDtypeStruct((LINR, 128), jnp.float32),
  )(lin_t)


def _sc_logits(iphys, ipar8, tphys, tpar, rphys, rpar, emb_r, lin_r):
  mesh = plsc.VectorSubcoreMesh(core_axis_name="c", subcore_axis_name="s")

  @functools.partial(
      pl.kernel,
      out_type=[
          jax.ShapeDtypeStruct((B * T,), jnp.float32),
          jax.ShapeDtypeStruct((B * T,), jnp.float32),
      ],
      mesh=mesh,
      scratch_types=[
          pltpu.VMEM((BPW,), jnp.int32),
          pltpu.VMEM((BPW * 16,), jnp.int32),
          pltpu.VMEM((BPW, 128), jnp.float32),
          pltpu.VMEM((BPW * T,), jnp.int32),
          pltpu.VMEM((BPW * T,), jnp.int32),
          pltpu.VMEM((BPW * T,), jnp.int32),
          pltpu.VMEM((BPW * T,), jnp.int32),
          pltpu.VMEM((2, ROWS, 128), jnp.float32),
          pltpu.VMEM((BPW * T,), jnp.float32),
          pltpu.VMEM((BPW * T,), jnp.float32),
          pltpu.SemaphoreType.DMA,
          pltpu.SemaphoreType.DMA,
          pltpu.SemaphoreType.DMA,
      ],
  )
  def k(iphys_h, ipar8_h, tphys_h, tpar_h, rphys_h, rpar_h, emb_h, lin_h,
        pos_h, neg_h,
        iidx, iparv, embv, tidx, tparv, ridx, rparv, rows2, posv, negv,
        sem0, sem1, sem_e):
    wid = lax.axis_index("s") * NC + lax.axis_index("c")
    base = wid * BPW

    pltpu.sync_copy(iphys_h.at[pl.ds(base, BPW)], iidx)
    emb_cp = pltpu.async_copy(emb_h.at[iidx], embv, sem_e)
    pltpu.sync_copy(ipar8_h.at[pl.ds(base * 16, BPW * 16)], iparv)
    pltpu.sync_copy(tphys_h.at[pl.ds(base * T, BPW * T)], tidx)
    pltpu.sync_copy(tpar_h.at[pl.ds(base * T, BPW * T)], tparv)
    pltpu.sync_copy(rphys_h.at[pl.ds(base * T, BPW * T)], ridx)
    pltpu.sync_copy(rpar_h.at[pl.ds(base * T, BPW * T)], rparv)
    emb_cp.wait()

    sems = (sem0, sem1)

    def start(idxref, g, slot):
      pltpu.async_copy(
          lin_h.at[idxref.at[pl.ds(g * ROWS, ROWS)]],
          rows2.at[slot], sems[slot])

    def wait(idxref, g, slot):
      pltpu.make_async_copy(
          lin_h.at[idxref.at[pl.ds(g * ROWS, ROWS)]],
          rows2.at[slot], sems[slot]).wait()

    lane_masks = [lax.iota(jnp.int32, LANES) == j for j in range(LANES)]
    perms = [lax.iota(jnp.int32, LANES) ^ sh for sh in (8, 4, 2, 1)]

    def compute(parref, outflat, g, slot):
      res = jnp.zeros((LANES,), jnp.float32)
      e = None
      pv = None
      for d in range(ROWS):
        cb, t = divmod(d, T)
        if t == 0:
          b_local = g * CB + cb
          epar = iparv[pl.ds(b_local * 16, LANES)][0]
          eoff = epar * EMB
          e = [embv[b_local, pl.ds(eoff + c * LANES, LANES)]
               for c in range(EC)]
        if d % LANES == 0:
          pv = parref[pl.ds(g * ROWS + d, LANES)]
        cpar = pv[d % LANES]
        coff = cpar * EMB
        acc = None
        for c in range(EC):
          ctx = rows2[slot, d, pl.ds(coff + c * LANES, LANES)]
          p_ = ctx * e[c]
          acc = p_ if acc is None else acc + p_
        for p in perms:
          acc = acc + jnp.take(acc, p)
        res = jnp.where(lane_masks[d % LANES], acc, res)
        if d % LANES == LANES - 1:
          outflat[pl.ds(g * ROWS + (d - LANES + 1), LANES)] = res

    def run_table(idxref, parref, outref):
      start(idxref, 0, 0)

      def body(i, carry):
        g0 = 2 * i

        @pl.when(g0 + 1 < NCHUNK)
        def _():
          start(idxref, g0 + 1, 1)

        wait(idxref, g0, 0)
        compute(parref, outref, g0, 0)

        @pl.when(g0 + 2 < NCHUNK)
        def _():
          start(idxref, g0 + 2, 0)

        @pl.when(g0 + 1 < NCHUNK)
        def _():
          wait(idxref, g0 + 1, 1)
          compute(parref, outref, g0 + 1, 1)

        return carry

      lax.fori_loop(0, NCHUNK // 2, body, 0)

    run_table(tidx, tparv, posv)
    run_table(ridx, rparv, negv)

    pltpu.sync_copy(posv, pos_h.at[pl.ds(base * T, BPW * T)])
    pltpu.sync_copy(negv, neg_h.at[pl.ds(base * T, BPW * T)])

  return k(iphys, ipar8, tphys, tpar, rphys, rpar, emb_r, lin_r)


def _tc_loss(pos, neg):
  def body(pos_ref, neg_ref, o_ref):
    p = jax.nn.sigmoid(pos_ref[...])
    n = jax.nn.sigmoid(neg_ref[...])
    pst = -jnp.mean(jnp.log(p))
    ngt = -jnp.mean(jnp.log(1.0 - n + 1e-3))
    o_ref[0, 0] = pst + ngt

  out = pl.pallas_call(
      body,
      out_shape=jax.ShapeDtypeStruct((1, 1), jnp.float32),
      in_specs=[
          pl.BlockSpec(memory_space=pltpu.VMEM),
          pl.BlockSpec(memory_space=pltpu.VMEM),
      ],
      out_specs=pl.BlockSpec(memory_space=pltpu.SMEM),
  )(pos, neg)
  return out[0, 0]


def kernel(inpt, trgs, rand, emb_table, lin_w):
  inpt = inpt.astype(jnp.int32)
  trgs = trgs.astype(jnp.int32)
  rand = rand.astype(jnp.int32)

  emb_r = emb_table.reshape(EMBR, 128)
  iphys = inpt >> 1
  ipar16 = jnp.broadcast_to((inpt & 1)[:, None], (B, 16)).reshape(-1)

  lin_r = _tc_relayout(lin_w.T)
  hb = RBLK // 2
  tphys = ((trgs // RBLK) * hb + (trgs % hb)).reshape(-1)
  tpar = ((trgs // hb) & 1).reshape(-1)
  rphys = ((rand // RBLK) * hb + (rand % hb)).reshape(-1)
  rpar = ((rand // hb) & 1).reshape(-1)

  pos, neg = _sc_logits(iphys, ipar16, tphys, tpar, rphys, rpar,
                        emb_r, lin_r)
  return _tc_loss(pos.reshape(B * T // 128, 128),
                  neg.reshape(B * T // 128, 128))


# speedup vs baseline: 4.2864x; 1.8829x over previous
import functools

import jax
import jax.numpy as jnp
from jax import lax
from jax.experimental import pallas as pl
from jax.experimental.pallas import tpu as pltpu
from jax.experimental.pallas import tpu_sc as plsc

VOC = 1000000
EMB = 64
B = 4096
T = 20

NC = 2
NS = 16
NW = NC * NS
BPW = B // NW
CB = 4
ROWS = CB * T
NCHUNK = BPW // CB
LANES = 16
EC = EMB // LANES

RBLK = 8192
NRBLK = (VOC + RBLK - 1) // RBLK
LINR = NRBLK * RBLK // 2
EMBR = VOC // 2


def _tc_relayout(lin_t):
  def body(a_ref, o_ref):
    ident = jnp.eye(EMB, dtype=jnp.float32)
    t = lax.dot_general(a_ref[...], ident, (((0,), (0,)), ((), ())),
                        preferred_element_type=jnp.float32)
    o_ref[:, 0:EMB] = t[0:RBLK // 2, :]
    o_ref[:, EMB:2 * EMB] = t[RBLK // 2:RBLK, :]

  return pl.pallas_call(
      body,
      grid=(NRBLK,),
      in_specs=[pl.BlockSpec((EMB, RBLK), lambda i: (0, i))],
      out_specs=pl.BlockSpec((RBLK // 2, 128), lambda i: (i, 0)),
      out_shape=jax.ShapeDtypeStruct((LINR, 128), jnp.float32),
  )(lin_t)


def _sc_logits(icol16, ioff16, tphys, tpar, rphys, rpar, embT, lin_r):
  mesh = plsc.VectorSubcoreMesh(core_axis_name="c", subcore_axis_name="s")

  @functools.partial(
      pl.kernel,
      out_type=[
          jax.ShapeDtypeStruct((B * T,), jnp.float32),
          jax.ShapeDtypeStruct((B * T,), jnp.float32),
      ],
      mesh=mesh,
      scratch_types=[
          pltpu.VMEM((BPW * 16,), jnp.int32),
          pltpu.VMEM((BPW * 16,), jnp.int32),
          pltpu.VMEM((2, EMB, 128), jnp.float32),
          pltpu.VMEM((BPW * EMB,), jnp.float32),
          pltpu.VMEM((BPW * T,), jnp.int32),
          pltpu.VMEM((BPW * T,), jnp.int32),
          pltpu.VMEM((BPW * T,), jnp.int32),
          pltpu.VMEM((BPW * T,), jnp.int32),
          pltpu.VMEM((2, ROWS, 128), jnp.float32),
          pltpu.VMEM((BPW * T,), jnp.float32),
          pltpu.VMEM((BPW * T,), jnp.float32),
          pltpu.SemaphoreType.DMA,
          pltpu.SemaphoreType.DMA,
          pltpu.SemaphoreType.DMA,
          pltpu.SemaphoreType.DMA,
      ],
  )
  def k(icol_h, ioff_h, tphys_h, tpar_h, rphys_h, rpar_h, embT_h, lin_h,
        pos_h, neg_h,
        icolv, ioffv, ebuf, embs, tidx, tparv, ridx, rparv, rows2,
        posv, negv, sem0, sem1, seme0, seme1):
    wid = lax.axis_index("s") * NC + lax.axis_index("c")
    base = wid * BPW

    pltpu.sync_copy(icol_h.at[pl.ds(base * 16, BPW * 16)], icolv)
    pltpu.sync_copy(ioff_h.at[pl.ds(base * 16, BPW * 16)], ioffv)
    pltpu.sync_copy(tphys_h.at[pl.ds(base * T, BPW * T)], tidx)
    pltpu.sync_copy(tpar_h.at[pl.ds(base * T, BPW * T)], tparv)
    pltpu.sync_copy(rphys_h.at[pl.ds(base * T, BPW * T)], ridx)
    pltpu.sync_copy(rpar_h.at[pl.ds(base * T, BPW * T)], rparv)

    lane_masks16 = [lax.iota(jnp.int32, LANES) == j for j in range(LANES)]
    esems = (seme0, seme1)

    def estart(b, slot):
      col = icolv[pl.ds(b * 16, LANES)][0]
      pltpu.async_copy(embT_h.at[:, pl.ds(col * 128, 128)],
                       ebuf.at[slot], esems[slot])

    def ewait(b, slot):
      col = icolv[pl.ds(b * 16, LANES)][0]
      pltpu.make_async_copy(embT_h.at[:, pl.ds(col * 128, 128)],
                            ebuf.at[slot], esems[slot]).wait()

    def eproc(b, slot):
      joff = ioffv[pl.ds(b * 16, LANES)][0]
      grp = (joff // LANES) * LANES
      bidx = jnp.full((LANES,), joff % LANES, jnp.int32)
      for cc in range(EC):
        e_cc = jnp.zeros((LANES,), jnp.float32)
        for lc in range(LANES):
          v = ebuf[slot, cc * LANES + lc, pl.ds(grp, LANES)]
          bv = jnp.take(v, bidx)
          e_cc = jnp.where(lane_masks16[lc], bv, e_cc)
        embs[pl.ds(b * EMB + cc * LANES, LANES)] = e_cc

    estart(0, 0)

    def ebody(i, carry):
      b0 = 2 * i

      @pl.when(b0 + 1 < BPW)
      def _():
        estart(b0 + 1, 1)

      ewait(b0, 0)
      eproc(b0, 0)

      @pl.when(b0 + 2 < BPW)
      def _():
        estart(b0 + 2, 0)

      @pl.when(b0 + 1 < BPW)
      def _():
        ewait(b0 + 1, 1)
        eproc(b0 + 1, 1)

      return carry

    lax.fori_loop(0, BPW // 2, ebody, 0)

    sems = (sem0, sem1)

    def start(idxref, g, slot):
      pltpu.async_copy(
          lin_h.at[idxref.at[pl.ds(g * ROWS, ROWS)]],
          rows2.at[slot], sems[slot])

    def wait(idxref, g, slot):
      pltpu.make_async_copy(
          lin_h.at[idxref.at[pl.ds(g * ROWS, ROWS)]],
          rows2.at[slot], sems[slot]).wait()

    lane_masks = [lax.iota(jnp.int32, LANES) == j for j in range(LANES)]
    perms = [lax.iota(jnp.int32, LANES) ^ sh for sh in (8, 4, 2, 1)]

    def compute(parref, outflat, g, slot):
      res = jnp.zeros((LANES,), jnp.float32)
      e = None
      pv = None
      for d in range(ROWS):
        cb, t = divmod(d, T)
        if t == 0:
          b_local = g * CB + cb
          e = [embs[pl.ds(b_local * EMB + c * LANES, LANES)]
               for c in range(EC)]
        if d % LANES == 0:
          pv = parref[pl.ds(g * ROWS + d, LANES)]
        cpar = pv[d % LANES]
        coff = cpar * EMB
        acc = None
        for c in range(EC):
          ctx = rows2[slot, d, pl.ds(coff + c * LANES, LANES)]
          p_ = ctx * e[c]
          acc = p_ if acc is None else acc + p_
        for p in perms:
          acc = acc + jnp.take(acc, p)
        res = jnp.where(lane_masks[d % LANES], acc, res)
        if d % LANES == LANES - 1:
          outflat[pl.ds(g * ROWS + (d - LANES + 1), LANES)] = res

    def run_table(idxref, parref, outref):
      start(idxref, 0, 0)

      def body(i, carry):
        g0 = 2 * i

        @pl.when(g0 + 1 < NCHUNK)
        def _():
          start(idxref, g0 + 1, 1)

        wait(idxref, g0, 0)
        compute(parref, outref, g0, 0)

        @pl.when(g0 + 2 < NCHUNK)
        def _():
          start(idxref, g0 + 2, 0)

        @pl.when(g0 + 1 < NCHUNK)
        def _():
          wait(idxref, g0 + 1, 1)
          compute(parref, outref, g0 + 1, 1)

        return carry

      lax.fori_loop(0, NCHUNK // 2, body, 0)

    run_table(tidx, tparv, posv)
    run_table(ridx, rparv, negv)

    pltpu.sync_copy(posv, pos_h.at[pl.ds(base * T, BPW * T)])
    pltpu.sync_copy(negv, neg_h.at[pl.ds(base * T, BPW * T)])

  return k(icol16, ioff16, tphys, tpar, rphys, rpar, embT, lin_r)


def _tc_loss(pos, neg):
  def body(pos_ref, neg_ref, o_ref):
    p = jax.nn.sigmoid(pos_ref[...])
    n = jax.nn.sigmoid(neg_ref[...])
    pst = -jnp.mean(jnp.log(p))
    ngt = -jnp.mean(jnp.log(1.0 - n + 1e-3))
    o_ref[0, 0] = pst + ngt

  out = pl.pallas_call(
      body,
      out_shape=jax.ShapeDtypeStruct((1, 1), jnp.float32),
      in_specs=[
          pl.BlockSpec(memory_space=pltpu.VMEM),
          pl.BlockSpec(memory_space=pltpu.VMEM),
      ],
      out_specs=pl.BlockSpec(memory_space=pltpu.SMEM),
  )(pos, neg)
  return out[0, 0]


def kernel(inpt, trgs, rand, emb_table, lin_w):
  inpt = inpt.astype(jnp.int32)
  trgs = trgs.astype(jnp.int32)
  rand = rand.astype(jnp.int32)

  embT = emb_table.T
  icol16 = jnp.broadcast_to((inpt // 128)[:, None], (B, 16)).reshape(-1)
  ioff16 = jnp.broadcast_to((inpt % 128)[:, None], (B, 16)).reshape(-1)

  lin_r = _tc_relayout(lin_w.T)
  hb = RBLK // 2
  tphys = ((trgs // RBLK) * hb + (trgs % hb)).reshape(-1)
  tpar = ((trgs // hb) & 1).reshape(-1)
  rphys = ((rand // RBLK) * hb + (rand % hb)).reshape(-1)
  rpar = ((rand // hb) & 1).reshape(-1)

  pos, neg = _sc_logits(icol16, ioff16, tphys, tpar, rphys, rpar,
                        embT, lin_r)
  return _tc_loss(pos.reshape(B * T // 128, 128),
                  neg.reshape(B * T // 128, 128))


# speedup vs baseline: 4.6428x; 1.0831x over previous
import functools

import jax
import jax.numpy as jnp
from jax import lax
from jax.experimental import pallas as pl
from jax.experimental.pallas import tpu as pltpu
from jax.experimental.pallas import tpu_sc as plsc

VOC = 1000000
EMB = 64
B = 4096
T = 20

NC = 2
NS = 16
NW = NC * NS
BPW = B // NW
CB = 4
ROWS = CB * T
NCHUNK = BPW // CB
LANES = 16
EC = EMB // LANES

RBLK = 16384
NRBLK = (VOC + RBLK - 1) // RBLK
LINR = NRBLK * RBLK // 2
EMBR = VOC // 2


def _tc_relayout(lin_t):
  def body(a_ref, o_ref):
    ident = jnp.eye(EMB, dtype=jnp.float32)
    t = lax.dot_general(a_ref[...], ident, (((0,), (0,)), ((), ())),
                        preferred_element_type=jnp.float32)
    o_ref[:, 0:EMB] = t[0:RBLK // 2, :]
    o_ref[:, EMB:2 * EMB] = t[RBLK // 2:RBLK, :]

  return pl.pallas_call(
      body,
      grid=(NRBLK,),
      in_specs=[pl.BlockSpec((EMB, RBLK), lambda i: (0, i))],
      out_specs=pl.BlockSpec((RBLK // 2, 128), lambda i: (i, 0)),
      out_shape=jax.ShapeDtypeStruct((LINR, 128), jnp.float32),
  )(lin_t)


def _sc_logits(icol16, ioff16, tphys, tpar, rphys, rpar, embT, lin_r):
  mesh = plsc.VectorSubcoreMesh(core_axis_name="c", subcore_axis_name="s")

  @functools.partial(
      pl.kernel,
      out_type=[
          jax.ShapeDtypeStruct((B * T,), jnp.float32),
          jax.ShapeDtypeStruct((B * T,), jnp.float32),
      ],
      mesh=mesh,
      scratch_types=[
          pltpu.VMEM((BPW * 16,), jnp.int32),
          pltpu.VMEM((BPW * 16,), jnp.int32),
          pltpu.VMEM((2, EMB, 128), jnp.float32),
          pltpu.VMEM((BPW * EMB,), jnp.float32),
          pltpu.VMEM((BPW * T,), jnp.int32),
          pltpu.VMEM((BPW * T,), jnp.int32),
          pltpu.VMEM((BPW * T,), jnp.int32),
          pltpu.VMEM((BPW * T,), jnp.int32),
          pltpu.VMEM((2, ROWS, 128), jnp.float32),
          pltpu.VMEM((BPW * T,), jnp.float32),
          pltpu.VMEM((BPW * T,), jnp.float32),
          pltpu.SemaphoreType.DMA,
          pltpu.SemaphoreType.DMA,
          pltpu.SemaphoreType.DMA,
          pltpu.SemaphoreType.DMA,
      ],
  )
  def k(icol_h, ioff_h, tphys_h, tpar_h, rphys_h, rpar_h, embT_h, lin_h,
        pos_h, neg_h,
        icolv, ioffv, ebuf, embs, tidx, tparv, ridx, rparv, rows2,
        posv, negv, sem0, sem1, seme0, seme1):
    wid = lax.axis_index("s") * NC + lax.axis_index("c")
    base = wid * BPW

    pltpu.sync_copy(icol_h.at[pl.ds(base * 16, BPW * 16)], icolv)
    pltpu.sync_copy(ioff_h.at[pl.ds(base * 16, BPW * 16)], ioffv)
    pltpu.sync_copy(tphys_h.at[pl.ds(base * T, BPW * T)], tidx)
    pltpu.sync_copy(tpar_h.at[pl.ds(base * T, BPW * T)], tparv)
    pltpu.sync_copy(rphys_h.at[pl.ds(base * T, BPW * T)], ridx)
    pltpu.sync_copy(rpar_h.at[pl.ds(base * T, BPW * T)], rparv)

    lane_masks16 = [lax.iota(jnp.int32, LANES) == j for j in range(LANES)]
    esems = (seme0, seme1)

    def estart(b, slot):
      col = icolv[pl.ds(b * 16, LANES)][0]
      pltpu.async_copy(embT_h.at[:, pl.ds(col * 128, 128)],
                       ebuf.at[slot], esems[slot])

    def ewait(b, slot):
      col = icolv[pl.ds(b * 16, LANES)][0]
      pltpu.make_async_copy(embT_h.at[:, pl.ds(col * 128, 128)],
                            ebuf.at[slot], esems[slot]).wait()

    def eproc(b, slot):
      joff = ioffv[pl.ds(b * 16, LANES)][0]
      grp = (joff // LANES) * LANES
      bidx = jnp.full((LANES,), joff % LANES, jnp.int32)
      for cc in range(EC):
        e_cc = jnp.zeros((LANES,), jnp.float32)
        for lc in range(LANES):
          v = ebuf[slot, cc * LANES + lc, pl.ds(grp, LANES)]
          bv = jnp.take(v, bidx)
          e_cc = jnp.where(lane_masks16[lc], bv, e_cc)
        embs[pl.ds(b * EMB + cc * LANES, LANES)] = e_cc

    estart(0, 0)

    def ebody(i, carry):
      b0 = 2 * i

      @pl.when(b0 + 1 < BPW)
      def _():
        estart(b0 + 1, 1)

      ewait(b0, 0)
      eproc(b0, 0)

      @pl.when(b0 + 2 < BPW)
      def _():
        estart(b0 + 2, 0)

      @pl.when(b0 + 1 < BPW)
      def _():
        ewait(b0 + 1, 1)
        eproc(b0 + 1, 1)

      return carry

    lax.fori_loop(0, BPW // 2, ebody, 0)

    sems = (sem0, sem1)

    def start(idxref, g, slot):
      pltpu.async_copy(
          lin_h.at[idxref.at[pl.ds(g * ROWS, ROWS)]],
          rows2.at[slot], sems[slot])

    def wait(idxref, g, slot):
      pltpu.make_async_copy(
          lin_h.at[idxref.at[pl.ds(g * ROWS, ROWS)]],
          rows2.at[slot], sems[slot]).wait()

    lane_masks = [lax.iota(jnp.int32, LANES) == j for j in range(LANES)]
    perms = [lax.iota(jnp.int32, LANES) ^ sh for sh in (8, 4, 2, 1)]

    def compute(parref, outflat, g, slot):
      res = jnp.zeros((LANES,), jnp.float32)
      e = None
      pv = None
      for d in range(ROWS):
        cb, t = divmod(d, T)
        if t == 0:
          b_local = g * CB + cb
          e = [embs[pl.ds(b_local * EMB + c * LANES, LANES)]
               for c in range(EC)]
        if d % LANES == 0:
          pv = parref[pl.ds(g * ROWS + d, LANES)]
        cpar = pv[d % LANES]
        coff = cpar * EMB
        acc = None
        for c in range(EC):
          ctx = rows2[slot, d, pl.ds(coff + c * LANES, LANES)]
          p_ = ctx * e[c]
          acc = p_ if acc is None else acc + p_
        for p in perms:
          acc = acc + jnp.take(acc, p)
        res = jnp.where(lane_masks[d % LANES], acc, res)
        if d % LANES == LANES - 1:
          outflat[pl.ds(g * ROWS + (d - LANES + 1), LANES)] = res

    def run_table(idxref, parref, outref):
      start(idxref, 0, 0)

      def body(i, carry):
        g0 = 2 * i

        @pl.when(g0 + 1 < NCHUNK)
        def _():
          start(idxref, g0 + 1, 1)

        wait(idxref, g0, 0)
        compute(parref, outref, g0, 0)

        @pl.when(g0 + 2 < NCHUNK)
        def _():
          start(idxref, g0 + 2, 0)

        @pl.when(g0 + 1 < NCHUNK)
        def _():
          wait(idxref, g0 + 1, 1)
          compute(parref, outref, g0 + 1, 1)

        return carry

      lax.fori_loop(0, NCHUNK // 2, body, 0)

    run_table(tidx, tparv, posv)
    run_table(ridx, rparv, negv)

    pltpu.sync_copy(posv, pos_h.at[pl.ds(base * T, BPW * T)])
    pltpu.sync_copy(negv, neg_h.at[pl.ds(base * T, BPW * T)])

  return k(icol16, ioff16, tphys, tpar, rphys, rpar, embT, lin_r)


def _tc_loss(pos, neg):
  def body(pos_ref, neg_ref, o_ref):
    p = jax.nn.sigmoid(pos_ref[...])
    n = jax.nn.sigmoid(neg_ref[...])
    pst = -jnp.mean(jnp.log(p))
    ngt = -jnp.mean(jnp.log(1.0 - n + 1e-3))
    o_ref[0, 0] = pst + ngt

  out = pl.pallas_call(
      body,
      out_shape=jax.ShapeDtypeStruct((1, 1), jnp.float32),
      in_specs=[
          pl.BlockSpec(memory_space=pltpu.VMEM),
          pl.BlockSpec(memory_space=pltpu.VMEM),
      ],
      out_specs=pl.BlockSpec(memory_space=pltpu.SMEM),
  )(pos, neg)
  return out[0, 0]


def kernel(inpt, trgs, rand, emb_table, lin_w):
  inpt = inpt.astype(jnp.int32)
  trgs = trgs.astype(jnp.int32)
  rand = rand.astype(jnp.int32)

  embT = emb_table.T
  icol16 = jnp.broadcast_to((inpt // 128)[:, None], (B, 16)).reshape(-1)
  ioff16 = jnp.broadcast_to((inpt % 128)[:, None], (B, 16)).reshape(-1)

  lin_r = _tc_relayout(lin_w.T)
  hb = RBLK // 2
  tphys = ((trgs // RBLK) * hb + (trgs % hb)).reshape(-1)
  tpar = ((trgs // hb) & 1).reshape(-1)
  rphys = ((rand // RBLK) * hb + (rand % hb)).reshape(-1)
  rpar = ((rand // hb) & 1).reshape(-1)

  pos, neg = _sc_logits(icol16, ioff16, tphys, tpar, rphys, rpar,
                        embT, lin_r)
  return _tc_loss(pos.reshape(B * T // 128, 128),
                  neg.reshape(B * T // 128, 128))


# speedup vs baseline: 4.8343x; 1.0412x over previous
import functools

import jax
import jax.numpy as jnp
from jax import lax
from jax.experimental import pallas as pl
from jax.experimental.pallas import tpu as pltpu
from jax.experimental.pallas import tpu_sc as plsc

VOC = 1000000
EMB = 64
B = 4096
T = 20

NC = 2
NS = 16
NW = NC * NS
BPW = B // NW
CB = 4
ROWS = CB * T
NCHUNK = BPW // CB
LANES = 16
EC = EMB // LANES

RBLK = 32768
NRBLK = (VOC + RBLK - 1) // RBLK
LINR = NRBLK * RBLK // 2
EMBR = VOC // 2


def _tc_relayout(lin_t):
  def body(a_ref, o_ref):
    ident = jnp.eye(EMB, dtype=jnp.float32)
    t = lax.dot_general(a_ref[...], ident, (((0,), (0,)), ((), ())),
                        preferred_element_type=jnp.float32)
    o_ref[:, 0:EMB] = t[0:RBLK // 2, :]
    o_ref[:, EMB:2 * EMB] = t[RBLK // 2:RBLK, :]

  return pl.pallas_call(
      body,
      grid=(NRBLK,),
      in_specs=[pl.BlockSpec((EMB, RBLK), lambda i: (0, i))],
      out_specs=pl.BlockSpec((RBLK // 2, 128), lambda i: (i, 0)),
      out_shape=jax.ShapeDtypeStruct((LINR, 128), jnp.float32),
  )(lin_t)


def _sc_logits(icol16, ioff16, tphys, tpar, rphys, rpar, embT, lin_r):
  mesh = plsc.VectorSubcoreMesh(core_axis_name="c", subcore_axis_name="s")

  @functools.partial(
      pl.kernel,
      out_type=[
          jax.ShapeDtypeStruct((B * T,), jnp.float32),
          jax.ShapeDtypeStruct((B * T,), jnp.float32),
      ],
      mesh=mesh,
      scratch_types=[
          pltpu.VMEM((BPW * 16,), jnp.int32),
          pltpu.VMEM((BPW * 16,), jnp.int32),
          pltpu.VMEM((2, EMB, 128), jnp.float32),
          pltpu.VMEM((BPW * EMB,), jnp.float32),
          pltpu.VMEM((BPW * T,), jnp.int32),
          pltpu.VMEM((BPW * T,), jnp.int32),
          pltpu.VMEM((BPW * T,), jnp.int32),
          pltpu.VMEM((BPW * T,), jnp.int32),
          pltpu.VMEM((2, ROWS, 128), jnp.float32),
          pltpu.VMEM((BPW * T,), jnp.float32),
          pltpu.VMEM((BPW * T,), jnp.float32),
          pltpu.SemaphoreType.DMA,
          pltpu.SemaphoreType.DMA,
          pltpu.SemaphoreType.DMA,
          pltpu.SemaphoreType.DMA,
      ],
  )
  def k(icol_h, ioff_h, tphys_h, tpar_h, rphys_h, rpar_h, embT_h, lin_h,
        pos_h, neg_h,
        icolv, ioffv, ebuf, embs, tidx, tparv, ridx, rparv, rows2,
        posv, negv, sem0, sem1, seme0, seme1):
    wid = lax.axis_index("s") * NC + lax.axis_index("c")
    base = wid * BPW

    pltpu.sync_copy(icol_h.at[pl.ds(base * 16, BPW * 16)], icolv)
    pltpu.sync_copy(ioff_h.at[pl.ds(base * 16, BPW * 16)], ioffv)
    pltpu.sync_copy(tphys_h.at[pl.ds(base * T, BPW * T)], tidx)
    pltpu.sync_copy(tpar_h.at[pl.ds(base * T, BPW * T)], tparv)
    pltpu.sync_copy(rphys_h.at[pl.ds(base * T, BPW * T)], ridx)
    pltpu.sync_copy(rpar_h.at[pl.ds(base * T, BPW * T)], rparv)

    lane_masks16 = [lax.iota(jnp.int32, LANES) == j for j in range(LANES)]
    esems = (seme0, seme1)

    def estart(b, slot):
      col = icolv[pl.ds(b * 16, LANES)][0]
      pltpu.async_copy(embT_h.at[:, pl.ds(col * 128, 128)],
                       ebuf.at[slot], esems[slot])

    def ewait(b, slot):
      col = icolv[pl.ds(b * 16, LANES)][0]
      pltpu.make_async_copy(embT_h.at[:, pl.ds(col * 128, 128)],
                            ebuf.at[slot], esems[slot]).wait()

    def eproc(b, slot):
      joff = ioffv[pl.ds(b * 16, LANES)][0]
      grp = (joff // LANES) * LANES
      bidx = jnp.full((LANES,), joff % LANES, jnp.int32)
      for cc in range(EC):
        e_cc = jnp.zeros((LANES,), jnp.float32)
        for lc in range(LANES):
          v = ebuf[slot, cc * LANES + lc, pl.ds(grp, LANES)]
          bv = jnp.take(v, bidx)
          e_cc = jnp.where(lane_masks16[lc], bv, e_cc)
        embs[pl.ds(b * EMB + cc * LANES, LANES)] = e_cc

    estart(0, 0)

    def ebody(i, carry):
      b0 = 2 * i

      @pl.when(b0 + 1 < BPW)
      def _():
        estart(b0 + 1, 1)

      ewait(b0, 0)
      eproc(b0, 0)

      @pl.when(b0 + 2 < BPW)
      def _():
        estart(b0 + 2, 0)

      @pl.when(b0 + 1 < BPW)
      def _():
        ewait(b0 + 1, 1)
        eproc(b0 + 1, 1)

      return carry

    lax.fori_loop(0, BPW // 2, ebody, 0)

    sems = (sem0, sem1)

    def start(idxref, g, slot):
      pltpu.async_copy(
          lin_h.at[idxref.at[pl.ds(g * ROWS, ROWS)]],
          rows2.at[slot], sems[slot])

    def wait(idxref, g, slot):
      pltpu.make_async_copy(
          lin_h.at[idxref.at[pl.ds(g * ROWS, ROWS)]],
          rows2.at[slot], sems[slot]).wait()

    lane_masks = [lax.iota(jnp.int32, LANES) == j for j in range(LANES)]
    perms = [lax.iota(jnp.int32, LANES) ^ sh for sh in (8, 4, 2, 1)]

    def compute(parref, outflat, g, slot):
      res = jnp.zeros((LANES,), jnp.float32)
      e = None
      pv = None
      for d in range(ROWS):
        cb, t = divmod(d, T)
        if t == 0:
          b_local = g * CB + cb
          e = [embs[pl.ds(b_local * EMB + c * LANES, LANES)]
               for c in range(EC)]
        if d % LANES == 0:
          pv = parref[pl.ds(g * ROWS + d, LANES)]
        cpar = pv[d % LANES]
        coff = cpar * EMB
        acc = None
        for c in range(EC):
          ctx = rows2[slot, d, pl.ds(coff + c * LANES, LANES)]
          p_ = ctx * e[c]
          acc = p_ if acc is None else acc + p_
        for p in perms:
          acc = acc + jnp.take(acc, p)
        res = jnp.where(lane_masks[d % LANES], acc, res)
        if d % LANES == LANES - 1:
          outflat[pl.ds(g * ROWS + (d - LANES + 1), LANES)] = res

    def run_table(idxref, parref, outref):
      start(idxref, 0, 0)

      def body(i, carry):
        g0 = 2 * i

        @pl.when(g0 + 1 < NCHUNK)
        def _():
          start(idxref, g0 + 1, 1)

        wait(idxref, g0, 0)
        compute(parref, outref, g0, 0)

        @pl.when(g0 + 2 < NCHUNK)
        def _():
          start(idxref, g0 + 2, 0)

        @pl.when(g0 + 1 < NCHUNK)
        def _():
          wait(idxref, g0 + 1, 1)
          compute(parref, outref, g0 + 1, 1)

        return carry

      lax.fori_loop(0, NCHUNK // 2, body, 0)

    run_table(tidx, tparv, posv)
    run_table(ridx, rparv, negv)

    pltpu.sync_copy(posv, pos_h.at[pl.ds(base * T, BPW * T)])
    pltpu.sync_copy(negv, neg_h.at[pl.ds(base * T, BPW * T)])

  return k(icol16, ioff16, tphys, tpar, rphys, rpar, embT, lin_r)


def _tc_loss(pos, neg):
  def body(pos_ref, neg_ref, o_ref):
    p = jax.nn.sigmoid(pos_ref[...])
    n = jax.nn.sigmoid(neg_ref[...])
    pst = -jnp.mean(jnp.log(p))
    ngt = -jnp.mean(jnp.log(1.0 - n + 1e-3))
    o_ref[0, 0] = pst + ngt

  out = pl.pallas_call(
      body,
      out_shape=jax.ShapeDtypeStruct((1, 1), jnp.float32),
      in_specs=[
          pl.BlockSpec(memory_space=pltpu.VMEM),
          pl.BlockSpec(memory_space=pltpu.VMEM),
      ],
      out_specs=pl.BlockSpec(memory_space=pltpu.SMEM),
  )(pos, neg)
  return out[0, 0]


def kernel(inpt, trgs, rand, emb_table, lin_w):
  inpt = inpt.astype(jnp.int32)
  trgs = trgs.astype(jnp.int32)
  rand = rand.astype(jnp.int32)

  embT = emb_table.T
  icol16 = jnp.broadcast_to((inpt // 128)[:, None], (B, 16)).reshape(-1)
  ioff16 = jnp.broadcast_to((inpt % 128)[:, None], (B, 16)).reshape(-1)

  lin_r = _tc_relayout(lin_w.T)
  hb = RBLK // 2
  tphys = ((trgs // RBLK) * hb + (trgs % hb)).reshape(-1)
  tpar = ((trgs // hb) & 1).reshape(-1)
  rphys = ((rand // RBLK) * hb + (rand % hb)).reshape(-1)
  rpar = ((rand // hb) & 1).reshape(-1)

  pos, neg = _sc_logits(icol16, ioff16, tphys, tpar, rphys, rpar,
                        embT, lin_r)
  return _tc_loss(pos.reshape(B * T // 128, 128),
                  neg.reshape(B * T // 128, 128))


# speedup vs baseline: 5.8040x; 1.2006x over previous
import functools

import jax
import jax.numpy as jnp
from jax import lax
from jax.experimental import pallas as pl
from jax.experimental.pallas import tpu as pltpu
from jax.experimental.pallas import tpu_sc as plsc

VOC = 1000000
EMB = 64
B = 4096
T = 20

NC = 2
NS = 16
NW = NC * NS
BPW = B // NW
CB = 4
ROWS = CB * T
NCHUNK = BPW // CB
LANES = 16
EC = EMB // LANES

RBLK = 32768
NRBLK = (VOC + RBLK - 1) // RBLK
LINR = NRBLK * RBLK // 2
EMBR = VOC // 2


def _tc_relayout(lin_t):
  def body(a_ref, o_ref):
    ident = jnp.eye(EMB, dtype=jnp.float32)
    t = lax.dot_general(a_ref[...], ident, (((0,), (0,)), ((), ())),
                        preferred_element_type=jnp.float32)
    o_ref[:, 0:EMB] = t[0:RBLK // 2, :]
    o_ref[:, EMB:2 * EMB] = t[RBLK // 2:RBLK, :]

  return pl.pallas_call(
      body,
      grid=(NRBLK,),
      in_specs=[pl.BlockSpec((EMB, RBLK), lambda i: (0, i))],
      out_specs=pl.BlockSpec((RBLK // 2, 128), lambda i: (i, 0)),
      out_shape=jax.ShapeDtypeStruct((LINR, 128), jnp.float32),
  )(lin_t)


def _sc_emb(icol16, ioff16, embT):
  mesh = plsc.VectorSubcoreMesh(core_axis_name="c", subcore_axis_name="s")

  @functools.partial(
      pl.kernel,
      out_type=[jax.ShapeDtypeStruct((B * EMB,), jnp.float32)],
      mesh=mesh,
      scratch_types=[
          pltpu.VMEM((BPW * 16,), jnp.int32),
          pltpu.VMEM((BPW * 16,), jnp.int32),
          pltpu.VMEM((2, EMB, 128), jnp.float32),
          pltpu.VMEM((BPW * EMB,), jnp.float32),
          pltpu.SemaphoreType.DMA,
          pltpu.SemaphoreType.DMA,
      ],
  )
  def ka(icol_h, ioff_h, embT_h, embs_h,
         icolv, ioffv, ebuf, embs, seme0, seme1):
    wid = lax.axis_index("s") * NC + lax.axis_index("c")
    base = wid * BPW

    pltpu.sync_copy(icol_h.at[pl.ds(base * 16, BPW * 16)], icolv)
    pltpu.sync_copy(ioff_h.at[pl.ds(base * 16, BPW * 16)], ioffv)

    lane_masks16 = [lax.iota(jnp.int32, LANES) == j for j in range(LANES)]
    esems = (seme0, seme1)

    def estart(b, slot):
      col = icolv[pl.ds(b * 16, LANES)][0]
      pltpu.async_copy(embT_h.at[:, pl.ds(col * 128, 128)],
                       ebuf.at[slot], esems[slot])

    def ewait(b, slot):
      col = icolv[pl.ds(b * 16, LANES)][0]
      pltpu.make_async_copy(embT_h.at[:, pl.ds(col * 128, 128)],
                            ebuf.at[slot], esems[slot]).wait()

    def eproc(b, slot):
      joff = ioffv[pl.ds(b * 16, LANES)][0]
      grp = (joff // LANES) * LANES
      bidx = jnp.full((LANES,), joff % LANES, jnp.int32)
      for cc in range(EC):
        e_cc = jnp.zeros((LANES,), jnp.float32)
        for lc in range(LANES):
          v = ebuf[slot, cc * LANES + lc, pl.ds(grp, LANES)]
          bv = jnp.take(v, bidx)
          e_cc = jnp.where(lane_masks16[lc], bv, e_cc)
        embs[pl.ds(b * EMB + cc * LANES, LANES)] = e_cc

    estart(0, 0)

    def ebody(i, carry):
      b0 = 2 * i

      @pl.when(b0 + 1 < BPW)
      def _():
        estart(b0 + 1, 1)

      ewait(b0, 0)
      eproc(b0, 0)

      @pl.when(b0 + 2 < BPW)
      def _():
        estart(b0 + 2, 0)

      @pl.when(b0 + 1 < BPW)
      def _():
        ewait(b0 + 1, 1)
        eproc(b0 + 1, 1)

      return carry

    lax.fori_loop(0, BPW // 2, ebody, 0)
    pltpu.sync_copy(embs, embs_h.at[pl.ds(base * EMB, BPW * EMB)])

  return ka(icol16, ioff16, embT)[0]


def _sc_logits(embs_all, tphys, tpar, rphys, rpar, lin_r):
  mesh = plsc.VectorSubcoreMesh(core_axis_name="c", subcore_axis_name="s")

  @functools.partial(
      pl.kernel,
      out_type=[
          jax.ShapeDtypeStruct((B * T,), jnp.float32),
          jax.ShapeDtypeStruct((B * T,), jnp.float32),
      ],
      mesh=mesh,
      scratch_types=[
          pltpu.VMEM((BPW * EMB,), jnp.float32),
          pltpu.VMEM((BPW * T,), jnp.int32),
          pltpu.VMEM((BPW * T,), jnp.int32),
          pltpu.VMEM((BPW * T,), jnp.int32),
          pltpu.VMEM((BPW * T,), jnp.int32),
          pltpu.VMEM((2, ROWS, 128), jnp.float32),
          pltpu.VMEM((BPW * T,), jnp.float32),
          pltpu.VMEM((BPW * T,), jnp.float32),
          pltpu.SemaphoreType.DMA,
          pltpu.SemaphoreType.DMA,
      ],
  )
  def k(embs_h, tphys_h, tpar_h, rphys_h, rpar_h, lin_h,
        pos_h, neg_h,
        embs, tidx, tparv, ridx, rparv, rows2,
        posv, negv, sem0, sem1):
    wid = lax.axis_index("s") * NC + lax.axis_index("c")
    base = wid * BPW

    pltpu.sync_copy(embs_h.at[pl.ds(base * EMB, BPW * EMB)], embs)
    pltpu.sync_copy(tphys_h.at[pl.ds(base * T, BPW * T)], tidx)
    pltpu.sync_copy(tpar_h.at[pl.ds(base * T, BPW * T)], tparv)
    pltpu.sync_copy(rphys_h.at[pl.ds(base * T, BPW * T)], ridx)
    pltpu.sync_copy(rpar_h.at[pl.ds(base * T, BPW * T)], rparv)

    sems = (sem0, sem1)

    def start(idxref, g, slot):
      pltpu.async_copy(
          lin_h.at[idxref.at[pl.ds(g * ROWS, ROWS)]],
          rows2.at[slot], sems[slot])

    def wait(idxref, g, slot):
      pltpu.make_async_copy(
          lin_h.at[idxref.at[pl.ds(g * ROWS, ROWS)]],
          rows2.at[slot], sems[slot]).wait()

    lane_masks = [lax.iota(jnp.int32, LANES) == j for j in range(LANES)]
    perms = [lax.iota(jnp.int32, LANES) ^ sh for sh in (8, 4, 2, 1)]

    def compute(parref, outflat, g, slot):
      res = jnp.zeros((LANES,), jnp.float32)
      e = None
      pv = None
      for d in range(ROWS):
        cb, t = divmod(d, T)
        if t == 0:
          b_local = g * CB + cb
          e = [embs[pl.ds(b_local * EMB + c * LANES, LANES)]
               for c in range(EC)]
        if d % LANES == 0:
          pv = parref[pl.ds(g * ROWS + d, LANES)]
        cpar = pv[d % LANES]
        coff = cpar * EMB
        acc = None
        for c in range(EC):
          ctx = rows2[slot, d, pl.ds(coff + c * LANES, LANES)]
          p_ = ctx * e[c]
          acc = p_ if acc is None else acc + p_
        for p in perms:
          acc = acc + jnp.take(acc, p)
        res = jnp.where(lane_masks[d % LANES], acc, res)
        if d % LANES == LANES - 1:
          outflat[pl.ds(g * ROWS + (d - LANES + 1), LANES)] = res

    def run_table(idxref, parref, outref):
      start(idxref, 0, 0)

      def body(i, carry):
        g0 = 2 * i

        @pl.when(g0 + 1 < NCHUNK)
        def _():
          start(idxref, g0 + 1, 1)

        wait(idxref, g0, 0)
        compute(parref, outref, g0, 0)

        @pl.when(g0 + 2 < NCHUNK)
        def _():
          start(idxref, g0 + 2, 0)

        @pl.when(g0 + 1 < NCHUNK)
        def _():
          wait(idxref, g0 + 1, 1)
          compute(parref, outref, g0 + 1, 1)

        return carry

      lax.fori_loop(0, NCHUNK // 2, body, 0)

    run_table(tidx, tparv, posv)
    run_table(ridx, rparv, negv)

    pltpu.sync_copy(posv, pos_h.at[pl.ds(base * T, BPW * T)])
    pltpu.sync_copy(negv, neg_h.at[pl.ds(base * T, BPW * T)])

  return k(embs_all, tphys, tpar, rphys, rpar, lin_r)


def _tc_loss(pos, neg):
  def body(pos_ref, neg_ref, o_ref):
    p = jax.nn.sigmoid(pos_ref[...])
    n = jax.nn.sigmoid(neg_ref[...])
    pst = -jnp.mean(jnp.log(p))
    ngt = -jnp.mean(jnp.log(1.0 - n + 1e-3))
    o_ref[0, 0] = pst + ngt

  out = pl.pallas_call(
      body,
      out_shape=jax.ShapeDtypeStruct((1, 1), jnp.float32),
      in_specs=[
          pl.BlockSpec(memory_space=pltpu.VMEM),
          pl.BlockSpec(memory_space=pltpu.VMEM),
      ],
      out_specs=pl.BlockSpec(memory_space=pltpu.SMEM),
  )(pos, neg)
  return out[0, 0]


def kernel(inpt, trgs, rand, emb_table, lin_w):
  inpt = inpt.astype(jnp.int32)
  trgs = trgs.astype(jnp.int32)
  rand = rand.astype(jnp.int32)

  embT = emb_table.T
  icol16 = jnp.broadcast_to((inpt // 128)[:, None], (B, 16)).reshape(-1)
  ioff16 = jnp.broadcast_to((inpt % 128)[:, None], (B, 16)).reshape(-1)

  lin_r = _tc_relayout(lin_w.T)
  hb = RBLK // 2
  tphys = ((trgs // RBLK) * hb + (trgs % hb)).reshape(-1)
  tpar = ((trgs // hb) & 1).reshape(-1)
  rphys = ((rand // RBLK) * hb + (rand % hb)).reshape(-1)
  rpar = ((rand // hb) & 1).reshape(-1)

  embs_all = _sc_emb(icol16, ioff16, embT)
  pos, neg = _sc_logits(embs_all, tphys, tpar, rphys, rpar, lin_r)
  return _tc_loss(pos.reshape(B * T // 128, 128),
                  neg.reshape(B * T // 128, 128))


# speedup vs baseline: 5.8040x; 1.0000x over previous
import functools

import jax
import jax.numpy as jnp
from jax import lax
from jax.experimental import pallas as pl
from jax.experimental.pallas import tpu as pltpu
from jax.experimental.pallas import tpu_sc as plsc

VOC = 1000000
EMB = 64
B = 4096
T = 20

NC = 2
NS = 16
NW = NC * NS
BPW = B // NW
CB = 4
ROWS = CB * T
NCHUNK = BPW // CB
LANES = 16
EC = EMB // LANES

RBLK = 32768
NRBLK = (VOC + RBLK - 1) // RBLK
LINR = NRBLK * RBLK // 2
EMBR = VOC // 2


def _tc_relayout(lin_t):
  def body(a_ref, o_ref):
    t = jnp.transpose(a_ref[...], (1, 0))
    o_ref[:, 0:EMB] = t[0:RBLK // 2, :]
    o_ref[:, EMB:2 * EMB] = t[RBLK // 2:RBLK, :]

  return pl.pallas_call(
      body,
      grid=(NRBLK,),
      in_specs=[pl.BlockSpec((EMB, RBLK), lambda i: (0, i))],
      out_specs=pl.BlockSpec((RBLK // 2, 128), lambda i: (i, 0)),
      out_shape=jax.ShapeDtypeStruct((LINR, 128), jnp.float32),
  )(lin_t)


def _sc_emb(icol16, ioff16, embT):
  mesh = plsc.VectorSubcoreMesh(core_axis_name="c", subcore_axis_name="s")

  @functools.partial(
      pl.kernel,
      out_type=[jax.ShapeDtypeStruct((B * EMB,), jnp.float32)],
      mesh=mesh,
      scratch_types=[
          pltpu.VMEM((BPW * 16,), jnp.int32),
          pltpu.VMEM((BPW * 16,), jnp.int32),
          pltpu.VMEM((2, EMB, 128), jnp.float32),
          pltpu.VMEM((BPW * EMB,), jnp.float32),
          pltpu.SemaphoreType.DMA,
          pltpu.SemaphoreType.DMA,
      ],
  )
  def ka(icol_h, ioff_h, embT_h, embs_h,
         icolv, ioffv, ebuf, embs, seme0, seme1):
    wid = lax.axis_index("s") * NC + lax.axis_index("c")
    base = wid * BPW

    pltpu.sync_copy(icol_h.at[pl.ds(base * 16, BPW * 16)], icolv)
    pltpu.sync_copy(ioff_h.at[pl.ds(base * 16, BPW * 16)], ioffv)

    lane_masks16 = [lax.iota(jnp.int32, LANES) == j for j in range(LANES)]
    esems = (seme0, seme1)

    def estart(b, slot):
      col = icolv[pl.ds(b * 16, LANES)][0]
      pltpu.async_copy(embT_h.at[:, pl.ds(col * 128, 128)],
                       ebuf.at[slot], esems[slot])

    def ewait(b, slot):
      col = icolv[pl.ds(b * 16, LANES)][0]
      pltpu.make_async_copy(embT_h.at[:, pl.ds(col * 128, 128)],
                            ebuf.at[slot], esems[slot]).wait()

    def eproc(b, slot):
      joff = ioffv[pl.ds(b * 16, LANES)][0]
      grp = (joff // LANES) * LANES
      bidx = jnp.full((LANES,), joff % LANES, jnp.int32)
      for cc in range(EC):
        e_cc = jnp.zeros((LANES,), jnp.float32)
        for lc in range(LANES):
          v = ebuf[slot, cc * LANES + lc, pl.ds(grp, LANES)]
          bv = jnp.take(v, bidx)
          e_cc = jnp.where(lane_masks16[lc], bv, e_cc)
        embs[pl.ds(b * EMB + cc * LANES, LANES)] = e_cc

    estart(0, 0)

    def ebody(i, carry):
      b0 = 2 * i

      @pl.when(b0 + 1 < BPW)
      def _():
        estart(b0 + 1, 1)

      ewait(b0, 0)
      eproc(b0, 0)

      @pl.when(b0 + 2 < BPW)
      def _():
        estart(b0 + 2, 0)

      @pl.when(b0 + 1 < BPW)
      def _():
        ewait(b0 + 1, 1)
        eproc(b0 + 1, 1)

      return carry

    lax.fori_loop(0, BPW // 2, ebody, 0)
    pltpu.sync_copy(embs, embs_h.at[pl.ds(base * EMB, BPW * EMB)])

  return ka(icol16, ioff16, embT)[0]


def _sc_logits(embs_all, tphys, tpar, rphys, rpar, lin_r):
  mesh = plsc.VectorSubcoreMesh(core_axis_name="c", subcore_axis_name="s")

  @functools.partial(
      pl.kernel,
      out_type=[
          jax.ShapeDtypeStruct((B * T,), jnp.float32),
          jax.ShapeDtypeStruct((B * T,), jnp.float32),
      ],
      mesh=mesh,
      scratch_types=[
          pltpu.VMEM((BPW * EMB,), jnp.float32),
          pltpu.VMEM((BPW * T,), jnp.int32),
          pltpu.VMEM((BPW * T,), jnp.int32),
          pltpu.VMEM((BPW * T,), jnp.int32),
          pltpu.VMEM((BPW * T,), jnp.int32),
          pltpu.VMEM((2, ROWS, 128), jnp.float32),
          pltpu.VMEM((BPW * T,), jnp.float32),
          pltpu.VMEM((BPW * T,), jnp.float32),
          pltpu.SemaphoreType.DMA,
          pltpu.SemaphoreType.DMA,
      ],
  )
  def k(embs_h, tphys_h, tpar_h, rphys_h, rpar_h, lin_h,
        pos_h, neg_h,
        embs, tidx, tparv, ridx, rparv, rows2,
        posv, negv, sem0, sem1):
    wid = lax.axis_index("s") * NC + lax.axis_index("c")
    base = wid * BPW

    pltpu.sync_copy(embs_h.at[pl.ds(base * EMB, BPW * EMB)], embs)
    pltpu.sync_copy(tphys_h.at[pl.ds(base * T, BPW * T)], tidx)
    pltpu.sync_copy(tpar_h.at[pl.ds(base * T, BPW * T)], tparv)
    pltpu.sync_copy(rphys_h.at[pl.ds(base * T, BPW * T)], ridx)
    pltpu.sync_copy(rpar_h.at[pl.ds(base * T, BPW * T)], rparv)

    sems = (sem0, sem1)

    def start(idxref, g, slot):
      pltpu.async_copy(
          lin_h.at[idxref.at[pl.ds(g * ROWS, ROWS)]],
          rows2.at[slot], sems[slot])

    def wait(idxref, g, slot):
      pltpu.make_async_copy(
          lin_h.at[idxref.at[pl.ds(g * ROWS, ROWS)]],
          rows2.at[slot], sems[slot]).wait()

    lane_masks = [lax.iota(jnp.int32, LANES) == j for j in range(LANES)]
    perms = [lax.iota(jnp.int32, LANES) ^ sh for sh in (8, 4, 2, 1)]

    def compute(parref, outflat, g, slot):
      res = jnp.zeros((LANES,), jnp.float32)
      e = None
      pv = None
      for d in range(ROWS):
        cb, t = divmod(d, T)
        if t == 0:
          b_local = g * CB + cb
          e = [embs[pl.ds(b_local * EMB + c * LANES, LANES)]
               for c in range(EC)]
        if d % LANES == 0:
          pv = parref[pl.ds(g * ROWS + d, LANES)]
        cpar = pv[d % LANES]
        coff = cpar * EMB
        acc = None
        for c in range(EC):
          ctx = rows2[slot, d, pl.ds(coff + c * LANES, LANES)]
          p_ = ctx * e[c]
          acc = p_ if acc is None else acc + p_
        for p in perms:
          acc = acc + jnp.take(acc, p)
        res = jnp.where(lane_masks[d % LANES], acc, res)
        if d % LANES == LANES - 1:
          outflat[pl.ds(g * ROWS + (d - LANES + 1), LANES)] = res

    def run_table(idxref, parref, outref):
      start(idxref, 0, 0)

      def body(i, carry):
        g0 = 2 * i

        @pl.when(g0 + 1 < NCHUNK)
        def _():
          start(idxref, g0 + 1, 1)

        wait(idxref, g0, 0)
        compute(parref, outref, g0, 0)

        @pl.when(g0 + 2 < NCHUNK)
        def _():
          start(idxref, g0 + 2, 0)

        @pl.when(g0 + 1 < NCHUNK)
        def _():
          wait(idxref, g0 + 1, 1)
          compute(parref, outref, g0 + 1, 1)

        return carry

      lax.fori_loop(0, NCHUNK // 2, body, 0)

    run_table(tidx, tparv, posv)
    run_table(ridx, rparv, negv)

    pltpu.sync_copy(posv, pos_h.at[pl.ds(base * T, BPW * T)])
    pltpu.sync_copy(negv, neg_h.at[pl.ds(base * T, BPW * T)])

  return k(embs_all, tphys, tpar, rphys, rpar, lin_r)


def _tc_loss(pos, neg):
  def body(pos_ref, neg_ref, o_ref):
    p = jax.nn.sigmoid(pos_ref[...])
    n = jax.nn.sigmoid(neg_ref[...])
    pst = -jnp.mean(jnp.log(p))
    ngt = -jnp.mean(jnp.log(1.0 - n + 1e-3))
    o_ref[0, 0] = pst + ngt

  out = pl.pallas_call(
      body,
      out_shape=jax.ShapeDtypeStruct((1, 1), jnp.float32),
      in_specs=[
          pl.BlockSpec(memory_space=pltpu.VMEM),
          pl.BlockSpec(memory_space=pltpu.VMEM),
      ],
      out_specs=pl.BlockSpec(memory_space=pltpu.SMEM),
  )(pos, neg)
  return out[0, 0]


def kernel(inpt, trgs, rand, emb_table, lin_w):
  inpt = inpt.astype(jnp.int32)
  trgs = trgs.astype(jnp.int32)
  rand = rand.astype(jnp.int32)

  embT = emb_table.T
  icol16 = jnp.broadcast_to((inpt // 128)[:, None], (B, 16)).reshape(-1)
  ioff16 = jnp.broadcast_to((inpt % 128)[:, None], (B, 16)).reshape(-1)

  lin_r = _tc_relayout(lin_w.T)
  hb = RBLK // 2
  tphys = ((trgs // RBLK) * hb + (trgs % hb)).reshape(-1)
  tpar = ((trgs // hb) & 1).reshape(-1)
  rphys = ((rand // RBLK) * hb + (rand % hb)).reshape(-1)
  rpar = ((rand // hb) & 1).reshape(-1)

  embs_all = _sc_emb(icol16, ioff16, embT)
  pos, neg = _sc_logits(embs_all, tphys, tpar, rphys, rpar, lin_r)
  return _tc_loss(pos.reshape(B * T // 128, 128),
                  neg.reshape(B * T // 128, 128))


# speedup vs baseline: 5.8067x; 1.0005x over previous
import functools

import jax
import jax.numpy as jnp
from jax import lax
from jax.experimental import pallas as pl
from jax.experimental.pallas import tpu as pltpu
from jax.experimental.pallas import tpu_sc as plsc

VOC = 1000000
EMB = 64
B = 4096
T = 20

NC = 2
NS = 16
NW = NC * NS
BPW = B // NW
CB = 4
ROWS = CB * T
NCHUNK = BPW // CB
LANES = 16
EC = EMB // LANES

RBLK = 32768
NRBLK = (VOC + RBLK - 1) // RBLK
LINR = NRBLK * RBLK // 2


def _tc_relayout(lin_t):
  def body(a_ref, o_ref):
    t = jnp.transpose(a_ref[...], (1, 0))
    o_ref[:, 0:EMB] = t[0:RBLK // 2, :]
    o_ref[:, EMB:2 * EMB] = t[RBLK // 2:RBLK, :]

  return pl.pallas_call(
      body,
      grid=(NRBLK,),
      in_specs=[pl.BlockSpec((EMB, RBLK), lambda i: (0, i))],
      out_specs=pl.BlockSpec((RBLK // 2, 128), lambda i: (i, 0)),
      out_shape=jax.ShapeDtypeStruct((LINR, 128), jnp.float32),
  )(lin_t)


def _sc_emb(icol16, ioff16, embT):
  mesh = plsc.VectorSubcoreMesh(core_axis_name="c", subcore_axis_name="s")

  @functools.partial(
      pl.kernel,
      out_type=[jax.ShapeDtypeStruct((B * EMB,), jnp.float32)],
      mesh=mesh,
      scratch_types=[
          pltpu.VMEM((BPW * 16,), jnp.int32),
          pltpu.VMEM((BPW * 16,), jnp.int32),
          pltpu.VMEM((2, EMB, 128), jnp.float32),
          pltpu.VMEM((BPW * EMB,), jnp.float32),
          pltpu.SemaphoreType.DMA,
          pltpu.SemaphoreType.DMA,
      ],
  )
  def ka(icol_h, ioff_h, embT_h, embs_h,
         icolv, ioffv, ebuf, embs, seme0, seme1):
    wid = lax.axis_index("s") * NC + lax.axis_index("c")
    base = wid * BPW

    pltpu.sync_copy(icol_h.at[pl.ds(base * 16, BPW * 16)], icolv)
    pltpu.sync_copy(ioff_h.at[pl.ds(base * 16, BPW * 16)], ioffv)

    lane_masks16 = [lax.iota(jnp.int32, LANES) == j for j in range(LANES)]
    esems = (seme0, seme1)

    def estart(b, slot):
      col = icolv[pl.ds(b * 16, LANES)][0]
      pltpu.async_copy(embT_h.at[:, pl.ds(col * 128, 128)],
                       ebuf.at[slot], esems[slot])

    def ewait(b, slot):
      col = icolv[pl.ds(b * 16, LANES)][0]
      pltpu.make_async_copy(embT_h.at[:, pl.ds(col * 128, 128)],
                            ebuf.at[slot], esems[slot]).wait()

    def eproc(b, slot):
      joff = ioffv[pl.ds(b * 16, LANES)][0]
      grp = (joff // LANES) * LANES
      bidx = jnp.full((LANES,), joff % LANES, jnp.int32)
      for cc in range(EC):
        e_cc = jnp.zeros((LANES,), jnp.float32)
        for lc in range(LANES):
          v = ebuf[slot, cc * LANES + lc, pl.ds(grp, LANES)]
          bv = jnp.take(v, bidx)
          e_cc = jnp.where(lane_masks16[lc], bv, e_cc)
        embs[pl.ds(b * EMB + cc * LANES, LANES)] = e_cc

    estart(0, 0)

    def ebody(i, carry):
      b0 = 2 * i

      @pl.when(b0 + 1 < BPW)
      def _():
        estart(b0 + 1, 1)

      ewait(b0, 0)
      eproc(b0, 0)

      @pl.when(b0 + 2 < BPW)
      def _():
        estart(b0 + 2, 0)

      @pl.when(b0 + 1 < BPW)
      def _():
        ewait(b0 + 1, 1)
        eproc(b0 + 1, 1)

      return carry

    lax.fori_loop(0, BPW // 2, ebody, 0)
    pltpu.sync_copy(embs, embs_h.at[pl.ds(base * EMB, BPW * EMB)])

  return ka(icol16, ioff16, embT)[0]


def _sc_logits(embs_all, tphys, tpar, rphys, rpar, lin_r):
  mesh = plsc.VectorSubcoreMesh(core_axis_name="c", subcore_axis_name="s")

  @functools.partial(
      pl.kernel,
      out_type=[
          jax.ShapeDtypeStruct((B * T,), jnp.float32),
          jax.ShapeDtypeStruct((B * T,), jnp.float32),
      ],
      mesh=mesh,
      scratch_types=[
          pltpu.VMEM((BPW * EMB,), jnp.float32),
          pltpu.VMEM((BPW * T,), jnp.int32),
          pltpu.VMEM((BPW * T,), jnp.int32),
          pltpu.VMEM((BPW * T,), jnp.int32),
          pltpu.VMEM((BPW * T,), jnp.int32),
          pltpu.VMEM((2, ROWS, 128), jnp.float32),
          pltpu.VMEM((BPW * T,), jnp.float32),
          pltpu.VMEM((BPW * T,), jnp.float32),
          pltpu.SemaphoreType.DMA,
          pltpu.SemaphoreType.DMA,
      ],
  )
  def k(embs_h, tphys_h, tpar_h, rphys_h, rpar_h, lin_h,
        pos_h, neg_h,
        embs, tidx, tparv, ridx, rparv, rows2,
        posv, negv, sem0, sem1):
    wid = lax.axis_index("s") * NC + lax.axis_index("c")
    base = wid * BPW

    pltpu.sync_copy(embs_h.at[pl.ds(base * EMB, BPW * EMB)], embs)
    pltpu.sync_copy(tphys_h.at[pl.ds(base * T, BPW * T)], tidx)
    pltpu.sync_copy(tpar_h.at[pl.ds(base * T, BPW * T)], tparv)
    pltpu.sync_copy(rphys_h.at[pl.ds(base * T, BPW * T)], ridx)
    pltpu.sync_copy(rpar_h.at[pl.ds(base * T, BPW * T)], rparv)

    sems = (sem0, sem1)

    def start(idxref, g, slot):
      pltpu.async_copy(
          lin_h.at[idxref.at[pl.ds(g * ROWS, ROWS)]],
          rows2.at[slot], sems[slot])

    def wait(idxref, g, slot):
      pltpu.make_async_copy(
          lin_h.at[idxref.at[pl.ds(g * ROWS, ROWS)]],
          rows2.at[slot], sems[slot]).wait()

    lane_masks = [lax.iota(jnp.int32, LANES) == j for j in range(LANES)]
    perms = [lax.iota(jnp.int32, LANES) ^ sh for sh in (8, 4, 2, 1)]

    def compute(parref, outflat, g, slot):
      res = jnp.zeros((LANES,), jnp.float32)
      e = None
      pv = None
      for d in range(ROWS):
        cb, t = divmod(d, T)
        if t == 0:
          b_local = g * CB + cb
          e = [embs[pl.ds(b_local * EMB + c * LANES, LANES)]
               for c in range(EC)]
        if d % LANES == 0:
          pv = parref[pl.ds(g * ROWS + d, LANES)]
        cpar = pv[d % LANES]
        coff = cpar * EMB
        acc = None
        for c in range(EC):
          ctx = rows2[slot, d, pl.ds(coff + c * LANES, LANES)]
          p_ = ctx * e[c]
          acc = p_ if acc is None else acc + p_
        for p in perms:
          acc = acc + jnp.take(acc, p)
        res = jnp.where(lane_masks[d % LANES], acc, res)
        if d % LANES == LANES - 1:
          outflat[pl.ds(g * ROWS + (d - LANES + 1), LANES)] = res

    def run_table(idxref, parref, outref):
      start(idxref, 0, 0)

      def body(i, carry):
        g0 = 2 * i

        @pl.when(g0 + 1 < NCHUNK)
        def _():
          start(idxref, g0 + 1, 1)

        wait(idxref, g0, 0)
        compute(parref, outref, g0, 0)

        @pl.when(g0 + 2 < NCHUNK)
        def _():
          start(idxref, g0 + 2, 0)

        @pl.when(g0 + 1 < NCHUNK)
        def _():
          wait(idxref, g0 + 1, 1)
          compute(parref, outref, g0 + 1, 1)

        return carry

      lax.fori_loop(0, NCHUNK // 2, body, 0)

    run_table(tidx, tparv, posv)
    run_table(ridx, rparv, negv)

    pltpu.sync_copy(posv, pos_h.at[pl.ds(base * T, BPW * T)])
    pltpu.sync_copy(negv, neg_h.at[pl.ds(base * T, BPW * T)])

  return k(embs_all, tphys, tpar, rphys, rpar, lin_r)


def _tc_loss(pos, neg):
  def body(pos_ref, neg_ref, o_ref):
    p = jax.nn.sigmoid(pos_ref[...])
    n = jax.nn.sigmoid(neg_ref[...])
    pst = -jnp.mean(jnp.log(p))
    ngt = -jnp.mean(jnp.log(1.0 - n + 1e-3))
    o_ref[0, 0] = pst + ngt

  out = pl.pallas_call(
      body,
      out_shape=jax.ShapeDtypeStruct((1, 1), jnp.float32),
      in_specs=[
          pl.BlockSpec(memory_space=pltpu.VMEM),
          pl.BlockSpec(memory_space=pltpu.VMEM),
      ],
      out_specs=pl.BlockSpec(memory_space=pltpu.SMEM),
  )(pos, neg)
  return out[0, 0]


def kernel(inpt, trgs, rand, emb_table, lin_w):
  inpt = inpt.astype(jnp.int32)
  trgs = trgs.astype(jnp.int32)
  rand = rand.astype(jnp.int32)

  embT = emb_table.T
  icol16 = jnp.broadcast_to((inpt // 128)[:, None], (B, 16)).reshape(-1)
  ioff16 = jnp.broadcast_to((inpt % 128)[:, None], (B, 16)).reshape(-1)

  lin_r = _tc_relayout(lin_w.T)
  hb = RBLK // 2
  tphys = ((trgs // RBLK) * hb + (trgs % hb)).reshape(-1)
  tpar = ((trgs // hb) & 1).reshape(-1)
  rphys = ((rand // RBLK) * hb + (rand % hb)).reshape(-1)
  rpar = ((rand // hb) & 1).reshape(-1)

  embs_all = _sc_emb(icol16, ioff16, embT)
  pos, neg = _sc_logits(embs_all, tphys, tpar, rphys, rpar, lin_r)
  return _tc_loss(pos.reshape(B * T // 128, 128),
                  neg.reshape(B * T // 128, 128))
